# baseline (speedup 1.0000x reference)
import functools

import jax
import jax.numpy as jnp
from jax import lax
from jax.experimental import pallas as pl
from jax.experimental.pallas import tpu as pltpu
from jax.experimental.pallas import tpu_sc as plsc

ATOM_DIMS = (119, 5, 12, 12, 10, 6, 6, 2, 2)
NF = len(ATOM_DIMS)
GROUP_DIMS = (640, 1440, 144)
NG = len(GROUP_DIMS)
H = 128
LANES = 16
NC, NS = 2, 16
NW = NC * NS
BLK = 128
NBLK = 25
PERW = BLK * NBLK
NSLOT = 3


def _encoder(xT_hbm, *rest):
    tabs_hbm = rest[:NG]
    out_hbm = rest[NG]
    tabs_spm = rest[NG + 1:2 * NG + 1]
    idx_v = rest[2 * NG + 1]
    fidx = rest[2 * NG + 2]
    acc = rest[2 * NG + 3]
    sem_stage = rest[2 * NG + 4]
    sem_idx = rest[2 * NG + 5]
    sem_g = rest[2 * NG + 6]
    sem_out = rest[2 * NG + 7]

    n = out_hbm.shape[0]
    cid = lax.axis_index("c")
    sid = lax.axis_index("s")
    wid = sid * NC + cid
    my_start = jnp.minimum(wid * PERW, n - PERW)

    for t in range(NF):
        pltpu.async_copy(
            xT_hbm.at[pl.ds(t * n + my_start, PERW)],
            idx_v.at[pl.ds(t * PERW, PERW)], sem_idx)

    @pl.when(sid < 8)
    def _stage_g0():
        s0 = sid * 80
        pltpu.async_copy(tabs_hbm[0].at[pl.ds(s0, 80), :],
                         tabs_spm[0].at[pl.ds(s0, 80), :], sem_stage)

    @pl.when(sid < 15)
    def _stage_g1():
        s1 = sid * 96
        pltpu.async_copy(tabs_hbm[1].at[pl.ds(s1, 96), :],
                         tabs_spm[1].at[pl.ds(s1, 96), :], sem_stage)

    @pl.when(sid == 8)
    def _stage_g2a():
        pltpu.async_copy(tabs_hbm[2].at[pl.ds(0, 64), :],
                         tabs_spm[2].at[pl.ds(0, 64), :], sem_stage)

    @pl.when(sid == 15)
    def _stage_g2b():
        pltpu.async_copy(tabs_hbm[2].at[pl.ds(64, 80), :],
                         tabs_spm[2].at[pl.ds(64, 80), :], sem_stage)

    for t in range(NF):
        pltpu.make_async_copy(
            xT_hbm.at[pl.ds(0, PERW)],
            idx_v.at[pl.ds(t * PERW, PERW)], sem_idx).wait()

    def fuse_body(j, _):
        def col(t):
            return idx_v[pl.ds(t * PERW + j * LANES, LANES)]
        f0 = col(0) * 5 + col(1)
        f1 = col(2) * 120 + col(3) * 10 + col(4)
        f2 = col(5) * 24 + col(6) * 4 + col(7) * 2 + col(8)
        fidx[pl.ds(0 * PERW + j * LANES, LANES)] = f0
        fidx[pl.ds(1 * PERW + j * LANES, LANES)] = f1
        fidx[pl.ds(2 * PERW + j * LANES, LANES)] = f2
        return 0

    lax.fori_loop(0, PERW // LANES, fuse_body, 0, unroll=False)

    @pl.when(sid < 8)
    def _wait_g0():
        pltpu.make_async_copy(tabs_hbm[0].at[pl.ds(0, 80), :],
                              tabs_spm[0].at[pl.ds(0, 80), :],
                              sem_stage).wait()

    @pl.when(sid < 15)
    def _wait_g1():
        pltpu.make_async_copy(tabs_hbm[1].at[pl.ds(0, 96), :],
                              tabs_spm[1].at[pl.ds(0, 96), :],
                              sem_stage).wait()

    @pl.when(sid == 8)
    def _wait_g2a():
        pltpu.make_async_copy(tabs_hbm[2].at[pl.ds(0, 64), :],
                              tabs_spm[2].at[pl.ds(0, 64), :],
                              sem_stage).wait()

    @pl.when(sid == 15)
    def _wait_g2b():
        pltpu.make_async_copy(tabs_hbm[2].at[pl.ds(0, 80), :],
                              tabs_spm[2].at[pl.ds(0, 80), :],
                              sem_stage).wait()

    plsc.subcore_barrier()

    def fire_gathers(k, slot):
        descs = [pltpu.async_copy(
            tabs_spm[0].at[fidx.at[pl.ds(k * BLK, BLK)]],
            acc.at[slot], sem_g)]
        descs += [
            pltpu.async_copy(
                tabs_spm[g].at[fidx.at[pl.ds(g * PERW + k * BLK, BLK)]],
                acc.at[slot], sem_g, add=True)
            for g in range(1, NG)
        ]
        return descs

    def wait_gathers(slot):
        pltpu.make_async_copy(
            tabs_spm[0].at[pl.ds(0, BLK)], acc.at[slot], sem_g).wait()
        for g in range(1, NG):
            pltpu.make_async_copy(
                tabs_spm[g].at[pl.ds(0, BLK)], acc.at[slot], sem_g).wait()

    fire_gathers(0, 0)

    def block_body(k, _):
        cur = lax.rem(k, NSLOT)
        nxt = lax.rem(k + 1, NSLOT)
        start = my_start + k * BLK

        @pl.when(k + 1 < NBLK)
        def _ahead():
            @pl.when(k >= NSLOT - 1)
            def _drain():
                pltpu.make_async_copy(
                    acc.at[nxt], out_hbm.at[pl.ds(0, BLK), :],
                    sem_out.at[nxt]).wait()
            fire_gathers(k + 1, nxt)

        wait_gathers(cur)
        pltpu.async_copy(
            acc.at[cur], out_hbm.at[pl.ds(start, BLK), :], sem_out.at[cur])
        return 0

    lax.fori_loop(0, NBLK, block_body, 0, unroll=False)

    for s in range(NSLOT):
        pltpu.make_async_copy(
            acc.at[s], out_hbm.at[pl.ds(0, BLK), :], sem_out.at[s]).wait()


def _fuse_tables(tables):
    t = tables
    g0 = (t[0][:, None, :] + t[1][None, :, :]).reshape(595, H)
    g0 = jnp.concatenate([g0, jnp.zeros((45, H), jnp.float32)])
    g1 = (t[2][:, None, None, :] + t[3][None, :, None, :]
          + t[4][None, None, :, :]).reshape(1440, H)
    g2 = (t[5][:, None, None, None, :] + t[6][None, :, None, None, :]
          + t[7][None, None, :, None, :]
          + t[8][None, None, None, :, :]).reshape(144, H)
    return g0, g1, g2


def kernel(x, tables):
    n = x.shape[0]
    xT = x.T.reshape(-1)
    fused = _fuse_tables(tables)

    mesh = plsc.VectorSubcoreMesh(
        core_axis_name="c", subcore_axis_name="s",
        num_cores=NC, num_subcores=NS,
    )
    run = functools.partial(
        pl.kernel,
        out_type=jax.ShapeDtypeStruct((n, H), jnp.float32),
        mesh=mesh,
        scratch_types=[
            *[pltpu.VMEM_SHARED((d, H), jnp.float32) for d in GROUP_DIMS],
            pltpu.VMEM((NF * PERW,), jnp.int32),
            pltpu.VMEM((NG * PERW,), jnp.int32),
            pltpu.VMEM((NSLOT, BLK, H), jnp.float32),
            pltpu.SemaphoreType.DMA,
            pltpu.SemaphoreType.DMA,
            pltpu.SemaphoreType.DMA,
            pltpu.SemaphoreType.DMA((NSLOT,)),
        ],
    )(_encoder)
    return run(xT, *fused)


# speedup vs baseline: 1.0121x; 1.0121x over previous
import functools

import jax
import jax.numpy as jnp
from jax import lax
from jax.experimental import pallas as pl
from jax.experimental.pallas import tpu as pltpu
from jax.experimental.pallas import tpu_sc as plsc

ATOM_DIMS = (119, 5, 12, 12, 10, 6, 6, 2, 2)
NF = len(ATOM_DIMS)
GROUP_DIMS = (640, 1440, 144)
NG = len(GROUP_DIMS)
H = 128
LANES = 16
NC, NS = 2, 16
NW = NC * NS
BLK = 128
NBLK = 25
PERW = BLK * NBLK
NSLOT = 3


def _encoder(xT_hbm, *rest):
    tabs_hbm = rest[:NG]
    out_hbm = rest[NG]
    tabs_spm = rest[NG + 1:2 * NG + 1]
    idx_v = rest[2 * NG + 1]
    fidx = rest[2 * NG + 2]
    acc = rest[2 * NG + 3]
    sem_stage = rest[2 * NG + 4]
    sem_idx = rest[2 * NG + 5]
    sem_g = rest[2 * NG + 6]
    sem_out = rest[2 * NG + 7]

    n = out_hbm.shape[0]
    cid = lax.axis_index("c")
    sid = lax.axis_index("s")
    wid = sid * NC + cid
    my_start = jnp.minimum(wid * PERW, n - PERW)

    for t in range(NF):
        pltpu.async_copy(
            xT_hbm.at[pl.ds(t * n + my_start, PERW)],
            idx_v.at[pl.ds(t * PERW, PERW)], sem_idx)

    @pl.when(sid < 8)
    def _stage_g0():
        s0 = sid * 80
        pltpu.async_copy(tabs_hbm[0].at[pl.ds(s0, 80), :],
                         tabs_spm[0].at[pl.ds(s0, 80), :], sem_stage)

    @pl.when(sid < 15)
    def _stage_g1():
        s1 = sid * 96
        pltpu.async_copy(tabs_hbm[1].at[pl.ds(s1, 96), :],
                         tabs_spm[1].at[pl.ds(s1, 96), :], sem_stage)

    @pl.when(sid == 8)
    def _stage_g2a():
        pltpu.async_copy(tabs_hbm[2].at[pl.ds(0, 64), :],
                         tabs_spm[2].at[pl.ds(0, 64), :], sem_stage)

    @pl.when(sid == 15)
    def _stage_g2b():
        pltpu.async_copy(tabs_hbm[2].at[pl.ds(64, 80), :],
                         tabs_spm[2].at[pl.ds(64, 80), :], sem_stage)

    for t in range(NF):
        pltpu.make_async_copy(
            xT_hbm.at[pl.ds(0, PERW)],
            idx_v.at[pl.ds(t * PERW, PERW)], sem_idx).wait()

    def fuse_block(b):
        for i in range(BLK // LANES):
            off = b * BLK + i * LANES

            def col(t):
                return idx_v[pl.ds(t * PERW + off, LANES)]
            f0 = col(0) * 5 + col(1)
            f1 = col(2) * 120 + col(3) * 10 + col(4)
            f2 = col(5) * 24 + col(6) * 4 + col(7) * 2 + col(8)
            fidx[pl.ds(0 * PERW + off, LANES)] = f0
            fidx[pl.ds(1 * PERW + off, LANES)] = f1
            fidx[pl.ds(2 * PERW + off, LANES)] = f2

    @pl.when(sid < 8)
    def _wait_g0():
        pltpu.make_async_copy(tabs_hbm[0].at[pl.ds(0, 80), :],
                              tabs_spm[0].at[pl.ds(0, 80), :],
                              sem_stage).wait()

    @pl.when(sid < 15)
    def _wait_g1():
        pltpu.make_async_copy(tabs_hbm[1].at[pl.ds(0, 96), :],
                              tabs_spm[1].at[pl.ds(0, 96), :],
                              sem_stage).wait()

    @pl.when(sid == 8)
    def _wait_g2a():
        pltpu.make_async_copy(tabs_hbm[2].at[pl.ds(0, 64), :],
                              tabs_spm[2].at[pl.ds(0, 64), :],
                              sem_stage).wait()

    @pl.when(sid == 15)
    def _wait_g2b():
        pltpu.make_async_copy(tabs_hbm[2].at[pl.ds(0, 80), :],
                              tabs_spm[2].at[pl.ds(0, 80), :],
                              sem_stage).wait()

    fuse_block(0)
    fuse_block(1)

    plsc.subcore_barrier()

    def fire_gathers(k, slot):
        descs = [pltpu.async_copy(
            tabs_spm[0].at[fidx.at[pl.ds(k * BLK, BLK)]],
            acc.at[slot], sem_g)]
        descs += [
            pltpu.async_copy(
                tabs_spm[g].at[fidx.at[pl.ds(g * PERW + k * BLK, BLK)]],
                acc.at[slot], sem_g, add=True)
            for g in range(1, NG)
        ]
        return descs

    def wait_gathers(slot):
        pltpu.make_async_copy(
            tabs_spm[0].at[pl.ds(0, BLK)], acc.at[slot], sem_g).wait()
        for g in range(1, NG):
            pltpu.make_async_copy(
                tabs_spm[g].at[pl.ds(0, BLK)], acc.at[slot], sem_g).wait()

    fire_gathers(0, 0)

    def block_body(k, _):
        cur = lax.rem(k, NSLOT)
        nxt = lax.rem(k + 1, NSLOT)
        start = my_start + k * BLK

        @pl.when(k + 1 < NBLK)
        def _ahead():
            @pl.when(k >= NSLOT - 1)
            def _drain():
                pltpu.make_async_copy(
                    acc.at[nxt], out_hbm.at[pl.ds(0, BLK), :],
                    sem_out.at[nxt]).wait()
            fire_gathers(k + 1, nxt)

        @pl.when(k + 2 < NBLK)
        def _fuse_ahead():
            fuse_block(k + 2)

        wait_gathers(cur)
        pltpu.async_copy(
            acc.at[cur], out_hbm.at[pl.ds(start, BLK), :], sem_out.at[cur])
        return 0

    lax.fori_loop(0, NBLK, block_body, 0, unroll=False)

    for s in range(NSLOT):
        pltpu.make_async_copy(
            acc.at[s], out_hbm.at[pl.ds(0, BLK), :], sem_out.at[s]).wait()


def _fuse_tables(tables):
    t = tables
    g0 = (t[0][:, None, :] + t[1][None, :, :]).reshape(595, H)
    g0 = jnp.concatenate([g0, jnp.zeros((45, H), jnp.float32)])
    g1 = (t[2][:, None, None, :] + t[3][None, :, None, :]
          + t[4][None, None, :, :]).reshape(1440, H)
    g2 = (t[5][:, None, None, None, :] + t[6][None, :, None, None, :]
          + t[7][None, None, :, None, :]
          + t[8][None, None, None, :, :]).reshape(144, H)
    return g0, g1, g2


def kernel(x, tables):
    n = x.shape[0]
    xT = x.T.reshape(-1)
    fused = _fuse_tables(tables)

    mesh = plsc.VectorSubcoreMesh(
        core_axis_name="c", subcore_axis_name="s",
        num_cores=NC, num_subcores=NS,
    )
    run = functools.partial(
        pl.kernel,
        out_type=jax.ShapeDtypeStruct((n, H), jnp.float32),
        mesh=mesh,
        scratch_types=[
            *[pltpu.VMEM_SHARED((d, H), jnp.float32) for d in GROUP_DIMS],
            pltpu.VMEM((NF * PERW,), jnp.int32),
            pltpu.VMEM((NG * PERW,), jnp.int32),
            pltpu.VMEM((NSLOT, BLK, H), jnp.float32),
            pltpu.SemaphoreType.DMA,
            pltpu.SemaphoreType.DMA,
            pltpu.SemaphoreType.DMA,
            pltpu.SemaphoreType.DMA((NSLOT,)),
        ],
    )(_encoder)
    return run(xT, *fused)
